# baseline (speedup 1.0000x reference)
import functools

import numpy as np
import jax
import jax.numpy as jnp
from jax import lax
from jax.experimental import pallas as pl
from jax.experimental.pallas import tpu as pltpu
from jax.experimental.pallas import tpu_sc as plsc

_K = 20
_INV = float(1.0 / np.sqrt(1.0 + 1e-5))
_BIG = np.float32(3.0e38)

_BQ = 128
_NL = 128
_D = 5
_BQH = 512
_NW = 32
_CH = 64


def _leaky(u):
    return jnp.maximum(u, 0.2 * u)


def _knn_body(N, xq_ref, xk_ref, xxq_ref, xxk_ref, pw_ref, qw_ref, qb_ref,
              idx_ref, p_ref, q_ref, d_ref):
    b = pl.program_id(0)
    ng = N // _NL
    xq = xq_ref[0]
    xxq = xxq_ref[0]
    p_ref[0] = jnp.dot(xq, pw_ref[...], preferred_element_type=jnp.float32)
    q_ref[0] = (jnp.dot(xq, qw_ref[...], preferred_element_type=jnp.float32)
                + qb_ref[...])

    M = [jnp.full((_BQ, _NL), _BIG, jnp.float32) for _ in range(_D)]
    G = [jnp.zeros((_BQ, _NL), jnp.int32) for _ in range(_D)]
    for g in range(ng):
        xkg = xk_ref[0, g * _NL:(g + 1) * _NL, :]
        inner = lax.dot_general(xq, xkg, (((1,), (1,)), ((), ())),
                                preferred_element_type=jnp.float32)
        c = (xxq + xxk_ref[0, :, g * _NL:(g + 1) * _NL]) - 2.0 * inner
        gi = jnp.int32(g)
        lt = [c < M[i] for i in range(_D)]
        for i in range(_D - 1, 0, -1):
            M[i] = jnp.where(lt[i], jnp.where(lt[i - 1], M[i - 1], c), M[i])
            G[i] = jnp.where(lt[i], jnp.where(lt[i - 1], G[i - 1], gi), G[i])
        M[0] = jnp.where(lt[0], c, M[0])
        G[0] = jnp.where(lt[0], gi, G[0])

    lanes = lax.broadcasted_iota(jnp.int32, (_BQ, _NL), 1)
    ibig = jnp.int32(1 << 30)
    ams = []
    ovm = jnp.zeros((_BQ, _NL), jnp.bool_)
    for _ in range(_K):
        m = jnp.min(M[0], axis=1, keepdims=True)
        col0 = G[0] * _NL + lanes
        amc = jnp.min(jnp.where(M[0] <= m, col0, ibig), axis=1, keepdims=True)
        ams.append(amc)
        win = col0 == amc
        for i in range(_D - 1):
            M[i] = jnp.where(win, M[i + 1], M[i])
            G[i] = jnp.where(win, G[i + 1], G[i])
        M[_D - 1] = jnp.where(win, _BIG, M[_D - 1])
        ovm = ovm | (win & (M[0] >= _BIG))
    ov = jnp.any(ovm)
    idx_ref[0] = jnp.concatenate(ams, axis=1) + b * N

    @pl.when(ov)
    def _fallback():
        inner_f = lax.dot_general(xq, xk_ref[0], (((1,), (1,)), ((), ())),
                                  preferred_element_type=jnp.float32)
        d_ref[...] = (xxq + xxk_ref[0]) - 2.0 * inner_f
        colbase = lax.broadcasted_iota(jnp.int32, (_BQ, N), 1)
        nbig = jnp.int32(N)
        fams = []
        for _ in range(_K):
            d = d_ref[...]
            fm = jnp.min(d, axis=1, keepdims=True)
            fam = jnp.min(jnp.where(d <= fm, colbase, nbig), axis=1,
                          keepdims=True)
            d_ref[...] = jnp.where(colbase == fam, _BIG, d)
            fams.append(fam)
        idx_ref[0] = jnp.concatenate(fams, axis=1) + b * N


def _knn_pq(xp, xxq, xxk, pw, qw, qb):
    B, N, CP = xp.shape
    Cout = pw.shape[1]
    nb = N // _BQ
    return pl.pallas_call(
        functools.partial(_knn_body, N),
        grid=(B, nb),
        in_specs=[
            pl.BlockSpec((1, _BQ, CP), lambda b, i: (b, i, 0)),
            pl.BlockSpec((1, N, CP), lambda b, i: (b, 0, 0)),
            pl.BlockSpec((1, _BQ, 1), lambda b, i: (b, i, 0)),
            pl.BlockSpec((1, 1, N), lambda b, i: (b, 0, 0)),
            pl.BlockSpec((CP, Cout), lambda b, i: (0, 0)),
            pl.BlockSpec((CP, Cout), lambda b, i: (0, 0)),
            pl.BlockSpec((1, Cout), lambda b, i: (0, 0)),
        ],
        out_specs=[
            pl.BlockSpec((1, _BQ, _K), lambda b, i: (b, i, 0)),
            pl.BlockSpec((1, _BQ, Cout), lambda b, i: (b, i, 0)),
            pl.BlockSpec((1, _BQ, Cout), lambda b, i: (b, i, 0)),
        ],
        out_shape=[
            jax.ShapeDtypeStruct((B, N, _K), jnp.int32),
            jax.ShapeDtypeStruct((B, N, Cout), jnp.float32),
            jax.ShapeDtypeStruct((B, N, Cout), jnp.float32),
        ],
        scratch_shapes=[pltpu.VMEM((_BQ, N), jnp.float32)],
    )(xp, xp, xxq, xxk, pw, qw, qb)


def _make_gather_max(TOT, Cout):
    mesh = plsc.VectorSubcoreMesh(core_axis_name="c", subcore_axis_name="s",
                                  num_cores=2, num_subcores=16)
    pwp = TOT // _NW
    nch = pwp // _CH
    cv = Cout // 16

    @functools.partial(
        pl.kernel,
        out_type=jax.ShapeDtypeStruct((TOT, Cout), jnp.float32),
        mesh=mesh,
        compiler_params=pltpu.CompilerParams(use_tc_tiling_on_sc=False),
        scratch_types=[
            pltpu.VMEM((_K, _CH), jnp.int32),
            pltpu.VMEM((_CH, Cout), jnp.float32),
            pltpu.VMEM((_CH, Cout), jnp.float32),
            pltpu.VMEM((_CH, Cout), jnp.float32),
            pltpu.VMEM((_CH, Cout), jnp.float32),
            pltpu.SemaphoreType.DMA,
            pltpu.SemaphoreType.DMA,
            pltpu.SemaphoreType.DMA,
        ],
    )
    def gm(p_hbm, q_hbm, idxt_hbm, out_hbm, idxm, rows0, rows1, accv, qv,
           sem0, sem1, semq):
        wid = lax.axis_index("s") * 2 + lax.axis_index("c")
        base = wid * pwp
        rows = (rows0, rows1)
        sems = (sem0, sem1)

        def accum(rv):
            def ploop(p, c3):
                for c in range(cv):
                    sl = pl.ds(c * 16, 16)
                    accv[p, sl] = jnp.maximum(accv[p, sl], rv[p, sl])
                return c3
            lax.fori_loop(0, _CH, ploop, 0)

        def chunk(ch, carry):
            pbase = base + ch * _CH
            qcp = pltpu.async_copy(q_hbm.at[pl.ds(pbase, _CH)], qv, semq)
            pltpu.sync_copy(idxt_hbm.at[:, pl.ds(pbase, _CH)], idxm)
            cp0 = pltpu.async_copy(p_hbm.at[idxm.at[0]], accv, sem0)
            cps = [None, pltpu.async_copy(p_hbm.at[idxm.at[1]], rows1, sem1)]
            cp0.wait()
            for j in range(2, _K):
                cps[j % 2] = pltpu.async_copy(p_hbm.at[idxm.at[j]],
                                              rows[j % 2], sems[j % 2])
                cps[(j - 1) % 2].wait()
                accum(rows[(j - 1) % 2])
            cps[(_K - 1) % 2].wait()
            accum(rows[(_K - 1) % 2])
            qcp.wait()

            def ploop2(p, c3):
                for c in range(cv):
                    sl = pl.ds(c * 16, 16)
                    u = accv[p, sl] + qv[p, sl]
                    accv[p, sl] = jnp.maximum(u, 0.2 * u)
                return c3
            lax.fori_loop(0, _CH, ploop2, 0)
            pltpu.sync_copy(accv, out_hbm.at[pl.ds(pbase, _CH)])
            return carry
        lax.fori_loop(0, nch, chunk, 0)

    return gm


def _make_gather_rows(TOT, Cp):
    mesh = plsc.VectorSubcoreMesh(core_axis_name="c", subcore_axis_name="s",
                                  num_cores=2, num_subcores=16)
    pwp = TOT // _NW
    nch = pwp // _CH

    @functools.partial(
        pl.kernel,
        out_type=jax.ShapeDtypeStruct((_K, TOT, Cp), jnp.float32),
        mesh=mesh,
        compiler_params=pltpu.CompilerParams(use_tc_tiling_on_sc=False),
        scratch_types=[
            pltpu.VMEM((_K, _CH), jnp.int32),
            pltpu.VMEM((_K, _CH, Cp), jnp.float32),
            pltpu.SemaphoreType.DMA,
        ],
    )
    def gr(x_hbm, idxt_hbm, out_hbm, idxm, buf, sem):
        wid = lax.axis_index("s") * 2 + lax.axis_index("c")
        base = wid * pwp

        def chunk(ch, carry):
            pbase = base + ch * _CH
            pltpu.sync_copy(idxt_hbm.at[:, pl.ds(pbase, _CH)], idxm)
            cps = [pltpu.async_copy(x_hbm.at[idxm.at[j]], buf.at[j], sem)
                   for j in range(_K)]
            for cp in cps:
                cp.wait()
            pltpu.sync_copy(buf, out_hbm.at[:, pl.ds(pbase, _CH), :])
            return carry
        lax.fori_loop(0, nch, chunk, 0)

    return gr


def _econv_body(C, xj_ref, xi_ref, w_ref, g_ref, b_ref, inv_ref, out_ref):
    xi = xi_ref[:, :C]
    inv = inv_ref[...]
    acc = None
    for j in range(_K):
        xjj = xj_ref[j][:, :C]
        e = jnp.concatenate([xjj - xi, xi], axis=1)
        y = lax.dot_general(e, w_ref[...], (((1,), (1,)), ((), ())),
                            preferred_element_type=jnp.float32)
        u = ((y * inv) * g_ref[...]) + b_ref[...]
        l = jnp.maximum(u, 0.2 * u)
        acc = l if acc is None else jnp.maximum(acc, l)
    out_ref[...] = acc


def _econv(xj, xp, W, g, b, inv_t, C):
    TOT, Cp = xp.shape
    Cout = W.shape[0]
    BP = 512
    return pl.pallas_call(
        functools.partial(_econv_body, C),
        grid=(TOT // BP,),
        in_specs=[
            pl.BlockSpec((_K, BP, Cp), lambda i: (0, i, 0)),
            pl.BlockSpec((BP, Cp), lambda i: (i, 0)),
            pl.BlockSpec((Cout, 2 * C), lambda i: (0, 0)),
            pl.BlockSpec((1, Cout), lambda i: (0, 0)),
            pl.BlockSpec((1, Cout), lambda i: (0, 0)),
            pl.BlockSpec((1, 1), lambda i: (0, 0)),
        ],
        out_specs=pl.BlockSpec((BP, Cout), lambda i: (i, 0)),
        out_shape=jax.ShapeDtypeStruct((TOT, Cout), jnp.float32),
    )(xj, xp, W, g[None, :], b[None, :], inv_t)


def _headA_body(xc_ref, wf_ref, bf_ref, gmax_ref):
    i = pl.program_id(1)
    f = _leaky(jnp.dot(xc_ref[0], wf_ref[...], preferred_element_type=jnp.float32)
               + bf_ref[...])
    bm = jnp.max(f, axis=0, keepdims=True)

    @pl.when(i == 0)
    def _():
        gmax_ref[0] = bm

    @pl.when(i != 0)
    def _():
        gmax_ref[0] = jnp.maximum(gmax_ref[0], bm)


def _headB_body(xc_ref, gm_ref, w1a_ref, w1b_ref, b1_ref, w2_ref, b2_ref,
                w3_ref, b3_ref, out_ref):
    xc = xc_ref[0]
    y1 = (jnp.dot(xc, w1a_ref[...], preferred_element_type=jnp.float32)
          + jnp.dot(gm_ref[0], w1b_ref[...], preferred_element_type=jnp.float32)
          + b1_ref[...])
    h1 = _leaky(y1)
    h2 = _leaky(jnp.dot(h1, w2_ref[...], preferred_element_type=jnp.float32)
                + b2_ref[...])
    out_ref[0] = (jnp.dot(h2, w3_ref[...], preferred_element_type=jnp.float32)
                  + b3_ref[...])


def _augment(x, CP):
    B, N, C = x.shape
    xx = jnp.sum(x * x, axis=-1)
    xp = x if C == CP else jnp.concatenate(
        [x, jnp.zeros((B, N, CP - C), jnp.float32)], axis=-1)
    return xp, xx[:, :, None], xx[:, None, :]


def _edge_weights(W, g, b, C, CP):
    Cout = W.shape[0]
    s = (_INV * g)[:, None]
    wd = W[:, :C] * s
    wq = (W[:, C:] - W[:, :C]) * s
    pw = jnp.zeros((CP, Cout), jnp.float32).at[:C, :].set(wd.T)
    qw = jnp.zeros((CP, Cout), jnp.float32).at[:C, :].set(wq.T)
    return pw, qw, b[None, :]


def _round8(n):
    return (n + 7) // 8 * 8


def kernel(xyz, W1, g1, b1, W2, g2, b2, W3, g3, b3, W4, g4, b4, Wf, gf, bf,
           Wh1, gh1, bh1, Wh2, gh2, bh2, Wh3, bh3):
    B, N, _ = xyz.shape
    TOT = B * N

    x = xyz
    feats = []
    inv_t = jnp.reshape(1.0 / jnp.sqrt(1.0 + 1e-5), (1, 1)).astype(jnp.float32)
    layers = ((W1, g1, b1), (W2, g2, b2), (W3, g3, b3), (W4, g4, b4))
    for li, (W, g, b) in enumerate(layers):
        C = x.shape[-1]
        CP = _round8(C)
        Cout = W.shape[0]
        xp, xxq, xxk = _augment(x, CP)
        pw, qw, qb = _edge_weights(W, g, b, C, CP)
        idx, P, Q = _knn_pq(xp, xxq, xxk, pw, qw, qb)
        idxt = idx.reshape(TOT, _K).T
        if li < 3:
            CpG = max(CP, 16)
            xg = xp.reshape(TOT, CP)
            if CpG != CP:
                xg = jnp.concatenate(
                    [xg, jnp.zeros((TOT, CpG - CP), jnp.float32)], axis=-1)
            xj = _make_gather_rows(TOT, CpG)(xg, idxt)
            xf = _econv(xj, xg, W, g, b, inv_t, C)
        else:
            xf = _make_gather_max(TOT, Cout)(
                P.reshape(TOT, Cout), Q.reshape(TOT, Cout), idxt)
        x = xf.reshape(B, N, Cout)
        feats.append(x)

    x_cat = jnp.concatenate(feats, axis=-1)
    emb = Wf.shape[0]
    Ccat = x_cat.shape[-1]

    wft = (Wf * (_INV * gf)[:, None]).T
    bfr = bf[None, :]
    w1s = Wh1 * (_INV * gh1)[:, None]
    w1at = w1s[:, :Ccat].T
    w1bt = w1s[:, Ccat:].T
    b1r = bh1[None, :]
    w2t = (Wh2 * (_INV * gh2)[:, None]).T
    b2r = bh2[None, :]
    w3t = Wh3.T
    b3r = bh3[None, :]

    nbh = N // _BQH
    gmax = pl.pallas_call(
        _headA_body,
        grid=(B, nbh),
        in_specs=[
            pl.BlockSpec((1, _BQH, Ccat), lambda b, i: (b, i, 0)),
            pl.BlockSpec((Ccat, emb), lambda b, i: (0, 0)),
            pl.BlockSpec((1, emb), lambda b, i: (0, 0)),
        ],
        out_specs=pl.BlockSpec((1, 1, emb), lambda b, i: (b, 0, 0)),
        out_shape=jax.ShapeDtypeStruct((B, 1, emb), jnp.float32),
    )(x_cat, wft, bfr)

    nc = Wh3.shape[0]
    h1dim = Wh1.shape[0]
    h2dim = Wh2.shape[0]
    logits = pl.pallas_call(
        _headB_body,
        grid=(B, nbh),
        in_specs=[
            pl.BlockSpec((1, _BQH, Ccat), lambda b, i: (b, i, 0)),
            pl.BlockSpec((1, 1, emb), lambda b, i: (b, 0, 0)),
            pl.BlockSpec((Ccat, h1dim), lambda b, i: (0, 0)),
            pl.BlockSpec((emb, h1dim), lambda b, i: (0, 0)),
            pl.BlockSpec((1, h1dim), lambda b, i: (0, 0)),
            pl.BlockSpec((h1dim, h2dim), lambda b, i: (0, 0)),
            pl.BlockSpec((1, h2dim), lambda b, i: (0, 0)),
            pl.BlockSpec((h2dim, nc), lambda b, i: (0, 0)),
            pl.BlockSpec((1, nc), lambda b, i: (0, 0)),
        ],
        out_specs=pl.BlockSpec((1, _BQH, nc), lambda b, i: (b, i, 0)),
        out_shape=jax.ShapeDtypeStruct((B, N, nc), jnp.float32),
    )(x_cat, gmax, w1at, w1bt, b1r, w2t, b2r, w3t, b3r)

    return logits


# speedup vs baseline: 1.2412x; 1.2412x over previous
import functools

import numpy as np
import jax
import jax.numpy as jnp
from jax import lax
from jax.experimental import pallas as pl
from jax.experimental.pallas import tpu as pltpu
from jax.experimental.pallas import tpu_sc as plsc

_K = 20
_INV = float(1.0 / np.sqrt(1.0 + 1e-5))
_BIG = np.float32(3.0e38)

_BQ = 256
_NL = 128
_D = 5
_BQH = 512
_NW = 32
_CH = 64


def _leaky(u):
    return jnp.maximum(u, 0.2 * u)


def _knn_body(N, xq_ref, xk_ref, xxq_ref, xxk_ref, pw_ref, qw_ref, qb_ref,
              idx_ref, p_ref, q_ref, d_ref):
    b = pl.program_id(0)
    ng = N // _NL
    xq = xq_ref[0]
    xxq = xxq_ref[0]
    p_ref[0] = jnp.dot(xq, pw_ref[...], preferred_element_type=jnp.float32)
    q_ref[0] = (jnp.dot(xq, qw_ref[...], preferred_element_type=jnp.float32)
                + qb_ref[...])

    M = [jnp.full((_BQ, _NL), _BIG, jnp.float32) for _ in range(_D)]
    G = [jnp.zeros((_BQ, _NL), jnp.int32) for _ in range(_D)]
    for g in range(ng):
        xkg = xk_ref[0, g * _NL:(g + 1) * _NL, :]
        inner = lax.dot_general(xq, xkg, (((1,), (1,)), ((), ())),
                                preferred_element_type=jnp.float32)
        c = (xxq + xxk_ref[0, :, g * _NL:(g + 1) * _NL]) - 2.0 * inner
        gi = jnp.int32(g)
        lt = [c < M[i] for i in range(_D)]
        for i in range(_D - 1, 0, -1):
            M[i] = jnp.where(lt[i], jnp.where(lt[i - 1], M[i - 1], c), M[i])
            G[i] = jnp.where(lt[i], jnp.where(lt[i - 1], G[i - 1], gi), G[i])
        M[0] = jnp.where(lt[0], c, M[0])
        G[0] = jnp.where(lt[0], gi, G[0])

    lanes = lax.broadcasted_iota(jnp.int32, (_BQ, _NL), 1)
    ibig = jnp.int32(1 << 30)
    ams = []
    ovm = jnp.zeros((_BQ, _NL), jnp.bool_)
    for _ in range(_K):
        m = jnp.min(M[0], axis=1, keepdims=True)
        col0 = G[0] * _NL + lanes
        amc = jnp.min(jnp.where(M[0] <= m, col0, ibig), axis=1, keepdims=True)
        ams.append(amc)
        win = col0 == amc
        for i in range(_D - 1):
            M[i] = jnp.where(win, M[i + 1], M[i])
            G[i] = jnp.where(win, G[i + 1], G[i])
        M[_D - 1] = jnp.where(win, _BIG, M[_D - 1])
        ovm = ovm | (win & (M[0] >= _BIG))
    ov = jnp.any(ovm)
    idx_ref[0] = jnp.concatenate(ams, axis=1) + b * N

    @pl.when(ov)
    def _fallback():
        inner_f = lax.dot_general(xq, xk_ref[0], (((1,), (1,)), ((), ())),
                                  preferred_element_type=jnp.float32)
        d_ref[...] = (xxq + xxk_ref[0]) - 2.0 * inner_f
        colbase = lax.broadcasted_iota(jnp.int32, (_BQ, N), 1)
        nbig = jnp.int32(N)
        fams = []
        for _ in range(_K):
            d = d_ref[...]
            fm = jnp.min(d, axis=1, keepdims=True)
            fam = jnp.min(jnp.where(d <= fm, colbase, nbig), axis=1,
                          keepdims=True)
            d_ref[...] = jnp.where(colbase == fam, _BIG, d)
            fams.append(fam)
        idx_ref[0] = jnp.concatenate(fams, axis=1) + b * N


def _knn_pq(xp, xxq, xxk, pw, qw, qb):
    B, N, CP = xp.shape
    Cout = pw.shape[1]
    nb = N // _BQ
    return pl.pallas_call(
        functools.partial(_knn_body, N),
        grid=(B, nb),
        in_specs=[
            pl.BlockSpec((1, _BQ, CP), lambda b, i: (b, i, 0)),
            pl.BlockSpec((1, N, CP), lambda b, i: (b, 0, 0)),
            pl.BlockSpec((1, _BQ, 1), lambda b, i: (b, i, 0)),
            pl.BlockSpec((1, 1, N), lambda b, i: (b, 0, 0)),
            pl.BlockSpec((CP, Cout), lambda b, i: (0, 0)),
            pl.BlockSpec((CP, Cout), lambda b, i: (0, 0)),
            pl.BlockSpec((1, Cout), lambda b, i: (0, 0)),
        ],
        out_specs=[
            pl.BlockSpec((1, _BQ, _K), lambda b, i: (b, i, 0)),
            pl.BlockSpec((1, _BQ, Cout), lambda b, i: (b, i, 0)),
            pl.BlockSpec((1, _BQ, Cout), lambda b, i: (b, i, 0)),
        ],
        out_shape=[
            jax.ShapeDtypeStruct((B, N, _K), jnp.int32),
            jax.ShapeDtypeStruct((B, N, Cout), jnp.float32),
            jax.ShapeDtypeStruct((B, N, Cout), jnp.float32),
        ],
        scratch_shapes=[pltpu.VMEM((_BQ, N), jnp.float32)],
    )(xp, xp, xxq, xxk, pw, qw, qb)


def _make_gather_max(TOT, Cout):
    mesh = plsc.VectorSubcoreMesh(core_axis_name="c", subcore_axis_name="s",
                                  num_cores=2, num_subcores=16)
    pwp = TOT // _NW
    nch = pwp // _CH
    cv = Cout // 16

    @functools.partial(
        pl.kernel,
        out_type=jax.ShapeDtypeStruct((TOT, Cout), jnp.float32),
        mesh=mesh,
        compiler_params=pltpu.CompilerParams(use_tc_tiling_on_sc=False),
        scratch_types=[
            pltpu.VMEM((_K, _CH), jnp.int32),
            pltpu.VMEM((_CH, Cout), jnp.float32),
            pltpu.VMEM((_CH, Cout), jnp.float32),
            pltpu.VMEM((_CH, Cout), jnp.float32),
            pltpu.VMEM((_CH, Cout), jnp.float32),
            pltpu.SemaphoreType.DMA,
            pltpu.SemaphoreType.DMA,
            pltpu.SemaphoreType.DMA,
        ],
    )
    def gm(p_hbm, q_hbm, idxt_hbm, out_hbm, idxm, rows0, rows1, accv, qv,
           sem0, sem1, semq):
        wid = lax.axis_index("s") * 2 + lax.axis_index("c")
        base = wid * pwp
        rows = (rows0, rows1)
        sems = (sem0, sem1)

        def accum(rv):
            def ploop(p, c3):
                for c in range(cv):
                    sl = pl.ds(c * 16, 16)
                    accv[p, sl] = jnp.maximum(accv[p, sl], rv[p, sl])
                return c3
            lax.fori_loop(0, _CH, ploop, 0)

        def chunk(ch, carry):
            pbase = base + ch * _CH
            qcp = pltpu.async_copy(q_hbm.at[pl.ds(pbase, _CH)], qv, semq)
            pltpu.sync_copy(idxt_hbm.at[:, pl.ds(pbase, _CH)], idxm)
            cp0 = pltpu.async_copy(p_hbm.at[idxm.at[0]], accv, sem0)
            cps = [None, pltpu.async_copy(p_hbm.at[idxm.at[1]], rows1, sem1)]
            cp0.wait()
            for j in range(2, _K):
                cps[j % 2] = pltpu.async_copy(p_hbm.at[idxm.at[j]],
                                              rows[j % 2], sems[j % 2])
                cps[(j - 1) % 2].wait()
                accum(rows[(j - 1) % 2])
            cps[(_K - 1) % 2].wait()
            accum(rows[(_K - 1) % 2])
            qcp.wait()

            def ploop2(p, c3):
                for c in range(cv):
                    sl = pl.ds(c * 16, 16)
                    u = accv[p, sl] + qv[p, sl]
                    accv[p, sl] = jnp.maximum(u, 0.2 * u)
                return c3
            lax.fori_loop(0, _CH, ploop2, 0)
            pltpu.sync_copy(accv, out_hbm.at[pl.ds(pbase, _CH)])
            return carry
        lax.fori_loop(0, nch, chunk, 0)

    return gm


def _make_gather_rows(TOT, Cp):
    mesh = plsc.VectorSubcoreMesh(core_axis_name="c", subcore_axis_name="s",
                                  num_cores=2, num_subcores=16)
    pwp = TOT // _NW
    nch = pwp // _CH

    @functools.partial(
        pl.kernel,
        out_type=jax.ShapeDtypeStruct((_K, TOT, Cp), jnp.float32),
        mesh=mesh,
        compiler_params=pltpu.CompilerParams(use_tc_tiling_on_sc=False),
        scratch_types=[
            pltpu.VMEM((_K, _CH), jnp.int32),
            pltpu.VMEM((_K, _CH, Cp), jnp.float32),
            pltpu.SemaphoreType.DMA,
        ],
    )
    def gr(x_hbm, idxt_hbm, out_hbm, idxm, buf, sem):
        wid = lax.axis_index("s") * 2 + lax.axis_index("c")
        base = wid * pwp

        def chunk(ch, carry):
            pbase = base + ch * _CH
            pltpu.sync_copy(idxt_hbm.at[:, pl.ds(pbase, _CH)], idxm)
            cps = [pltpu.async_copy(x_hbm.at[idxm.at[j]], buf.at[j], sem)
                   for j in range(_K)]
            for cp in cps:
                cp.wait()
            pltpu.sync_copy(buf, out_hbm.at[:, pl.ds(pbase, _CH), :])
            return carry
        lax.fori_loop(0, nch, chunk, 0)

    return gr


def _econv_body(C, xj_ref, xi_ref, w_ref, g_ref, b_ref, inv_ref, out_ref):
    xi = xi_ref[:, :C]
    inv = inv_ref[...]
    acc = None
    for j in range(_K):
        xjj = xj_ref[j][:, :C]
        e = jnp.concatenate([xjj - xi, xi], axis=1)
        y = lax.dot_general(e, w_ref[...], (((1,), (1,)), ((), ())),
                            preferred_element_type=jnp.float32)
        u = ((y * inv) * g_ref[...]) + b_ref[...]
        l = jnp.maximum(u, 0.2 * u)
        acc = l if acc is None else jnp.maximum(acc, l)
    out_ref[...] = acc


def _econv(xj, xp, W, g, b, inv_t, C):
    TOT, Cp = xp.shape
    Cout = W.shape[0]
    BP = 512
    return pl.pallas_call(
        functools.partial(_econv_body, C),
        grid=(TOT // BP,),
        in_specs=[
            pl.BlockSpec((_K, BP, Cp), lambda i: (0, i, 0)),
            pl.BlockSpec((BP, Cp), lambda i: (i, 0)),
            pl.BlockSpec((Cout, 2 * C), lambda i: (0, 0)),
            pl.BlockSpec((1, Cout), lambda i: (0, 0)),
            pl.BlockSpec((1, Cout), lambda i: (0, 0)),
            pl.BlockSpec((1, 1), lambda i: (0, 0)),
        ],
        out_specs=pl.BlockSpec((BP, Cout), lambda i: (i, 0)),
        out_shape=jax.ShapeDtypeStruct((TOT, Cout), jnp.float32),
    )(xj, xp, W, g[None, :], b[None, :], inv_t)


def _headA_body(xc_ref, wf_ref, bf_ref, gmax_ref):
    i = pl.program_id(1)
    f = _leaky(jnp.dot(xc_ref[0], wf_ref[...], preferred_element_type=jnp.float32)
               + bf_ref[...])
    bm = jnp.max(f, axis=0, keepdims=True)

    @pl.when(i == 0)
    def _():
        gmax_ref[0] = bm

    @pl.when(i != 0)
    def _():
        gmax_ref[0] = jnp.maximum(gmax_ref[0], bm)


def _headB_body(xc_ref, gm_ref, w1a_ref, w1b_ref, b1_ref, w2_ref, b2_ref,
                w3_ref, b3_ref, out_ref):
    xc = xc_ref[0]
    y1 = (jnp.dot(xc, w1a_ref[...], preferred_element_type=jnp.float32)
          + jnp.dot(gm_ref[0], w1b_ref[...], preferred_element_type=jnp.float32)
          + b1_ref[...])
    h1 = _leaky(y1)
    h2 = _leaky(jnp.dot(h1, w2_ref[...], preferred_element_type=jnp.float32)
                + b2_ref[...])
    out_ref[0] = (jnp.dot(h2, w3_ref[...], preferred_element_type=jnp.float32)
                  + b3_ref[...])


def _augment(x, CP):
    B, N, C = x.shape
    xx = jnp.sum(x * x, axis=-1)
    xp = x if C == CP else jnp.concatenate(
        [x, jnp.zeros((B, N, CP - C), jnp.float32)], axis=-1)
    return xp, xx[:, :, None], xx[:, None, :]


def _edge_weights(W, g, b, C, CP):
    Cout = W.shape[0]
    s = (_INV * g)[:, None]
    wd = W[:, :C] * s
    wq = (W[:, C:] - W[:, :C]) * s
    pw = jnp.zeros((CP, Cout), jnp.float32).at[:C, :].set(wd.T)
    qw = jnp.zeros((CP, Cout), jnp.float32).at[:C, :].set(wq.T)
    return pw, qw, b[None, :]


def _round8(n):
    return (n + 7) // 8 * 8


def kernel(xyz, W1, g1, b1, W2, g2, b2, W3, g3, b3, W4, g4, b4, Wf, gf, bf,
           Wh1, gh1, bh1, Wh2, gh2, bh2, Wh3, bh3):
    B, N, _ = xyz.shape
    TOT = B * N

    x = xyz
    feats = []
    inv_t = jnp.reshape(1.0 / jnp.sqrt(1.0 + 1e-5), (1, 1)).astype(jnp.float32)
    layers = ((W1, g1, b1), (W2, g2, b2), (W3, g3, b3), (W4, g4, b4))
    for li, (W, g, b) in enumerate(layers):
        C = x.shape[-1]
        CP = _round8(C)
        Cout = W.shape[0]
        xp, xxq, xxk = _augment(x, CP)
        pw, qw, qb = _edge_weights(W, g, b, C, CP)
        idx, P, Q = _knn_pq(xp, xxq, xxk, pw, qw, qb)
        idxt = idx.reshape(TOT, _K).T
        if li < 3:
            CpG = max(CP, 16)
            xg = xp.reshape(TOT, CP)
            if CpG != CP:
                xg = jnp.concatenate(
                    [xg, jnp.zeros((TOT, CpG - CP), jnp.float32)], axis=-1)
            xj = _make_gather_rows(TOT, CpG)(xg, idxt)
            xf = _econv(xj, xg, W, g, b, inv_t, C)
        else:
            xf = _make_gather_max(TOT, Cout)(
                P.reshape(TOT, Cout), Q.reshape(TOT, Cout), idxt)
        x = xf.reshape(B, N, Cout)
        feats.append(x)

    x_cat = jnp.concatenate(feats, axis=-1)
    emb = Wf.shape[0]
    Ccat = x_cat.shape[-1]

    wft = (Wf * (_INV * gf)[:, None]).T
    bfr = bf[None, :]
    w1s = Wh1 * (_INV * gh1)[:, None]
    w1at = w1s[:, :Ccat].T
    w1bt = w1s[:, Ccat:].T
    b1r = bh1[None, :]
    w2t = (Wh2 * (_INV * gh2)[:, None]).T
    b2r = bh2[None, :]
    w3t = Wh3.T
    b3r = bh3[None, :]

    nbh = N // _BQH
    gmax = pl.pallas_call(
        _headA_body,
        grid=(B, nbh),
        in_specs=[
            pl.BlockSpec((1, _BQH, Ccat), lambda b, i: (b, i, 0)),
            pl.BlockSpec((Ccat, emb), lambda b, i: (0, 0)),
            pl.BlockSpec((1, emb), lambda b, i: (0, 0)),
        ],
        out_specs=pl.BlockSpec((1, 1, emb), lambda b, i: (b, 0, 0)),
        out_shape=jax.ShapeDtypeStruct((B, 1, emb), jnp.float32),
    )(x_cat, wft, bfr)

    nc = Wh3.shape[0]
    h1dim = Wh1.shape[0]
    h2dim = Wh2.shape[0]
    logits = pl.pallas_call(
        _headB_body,
        grid=(B, nbh),
        in_specs=[
            pl.BlockSpec((1, _BQH, Ccat), lambda b, i: (b, i, 0)),
            pl.BlockSpec((1, 1, emb), lambda b, i: (b, 0, 0)),
            pl.BlockSpec((Ccat, h1dim), lambda b, i: (0, 0)),
            pl.BlockSpec((emb, h1dim), lambda b, i: (0, 0)),
            pl.BlockSpec((1, h1dim), lambda b, i: (0, 0)),
            pl.BlockSpec((h1dim, h2dim), lambda b, i: (0, 0)),
            pl.BlockSpec((1, h2dim), lambda b, i: (0, 0)),
            pl.BlockSpec((h2dim, nc), lambda b, i: (0, 0)),
            pl.BlockSpec((1, nc), lambda b, i: (0, 0)),
        ],
        out_specs=pl.BlockSpec((1, _BQH, nc), lambda b, i: (b, i, 0)),
        out_shape=jax.ShapeDtypeStruct((B, N, nc), jnp.float32),
    )(x_cat, gmax, w1at, w1bt, b1r, w2t, b2r, w3t, b3r)

    return logits
